# baseline (speedup 1.0000x reference)
import functools

import jax
import jax.numpy as jnp
from jax import lax
from jax.experimental import pallas as pl
from jax.experimental.pallas import tpu as pltpu


def _rnn_lm_kernel(sent_ref,
                   emb_ref, tgt_ref, len3_ref, lencol_ref,
                   wx0_ref, wh0_ref, b0_ref, wx1_ref, wh1_ref, b1_ref,
                   wl_ref, bl_ref,
                   logits_ref, pex_ref,
                   h0_ref, h1_ref, acc_ref, xall_ref, *, ts, bb):
    b = pl.program_id(0)
    t = pl.program_id(1)
    E = emb_ref.shape[-1]
    H = wh0_ref.shape[0]
    V = wl_ref.shape[1]

    @pl.when(t == 0)
    def _():
        h0_ref[...] = jnp.zeros_like(h0_ref)
        h1_ref[...] = jnp.zeros_like(h1_ref)
        acc_ref[...] = jnp.zeros_like(acc_ref)

    for k in range(ts):
        for i in range(bb):
            tok = sent_ref[b * bb + i, t * ts + k]
            xall_ref[k * bb + i, :] = emb_ref[tok, :]

    xp0 = (jnp.dot(xall_ref[...], wx0_ref[...],
                   preferred_element_type=jnp.float32)
           + b0_ref[...]).reshape(ts, bb, H)

    h0 = h0_ref[...]
    h1 = h1_ref[...]
    hs = []
    for k in range(ts):
        h0 = jnp.tanh(
            xp0[k] + jnp.dot(h0, wh0_ref[...],
                             preferred_element_type=jnp.float32))
        h1 = jnp.tanh(
            jnp.dot(h0, wx1_ref[...], preferred_element_type=jnp.float32)
            + jnp.dot(h1, wh1_ref[...], preferred_element_type=jnp.float32)
            + b1_ref[...])
        hs.append(h1)
    h0_ref[...] = h0
    h1_ref[...] = h1

    h_flat = jnp.swapaxes(jnp.stack(hs, axis=0), 0, 1).reshape(bb * ts, H)

    logits_ref[...] = (jnp.dot(h_flat.astype(jnp.bfloat16), wl_ref[...],
                               preferred_element_type=jnp.float32)
                       + bl_ref[...]).reshape(bb, ts, V)
    logits3 = logits_ref[...]

    tgt = tgt_ref[0, 0]
    vs = min(512, V)
    s = jnp.zeros((bb, ts, 1), jnp.float32)
    g = jnp.zeros((bb, ts, 1), jnp.float32)
    for j in range(V // vs):
        blk = logits3[:, :, j * vs:(j + 1) * vs]
        s = s + jnp.sum(jnp.exp(blk), axis=-1, keepdims=True)
        col = lax.broadcasted_iota(jnp.int32, (bb, ts, vs), 2) + j * vs
        g = g + jnp.sum(jnp.where(col == tgt, blk, 0.0),
                        axis=-1, keepdims=True)
    lse = jnp.log(s)

    k_ids = lax.broadcasted_iota(jnp.int32, (bb, ts, 1), 1)
    step_f = (k_ids + t * ts).astype(jnp.float32)
    masked = jnp.where(step_f < len3_ref[0], g - lse, 0.0)
    acc_ref[...] = acc_ref[...] + jnp.sum(masked, axis=1)

    @pl.when(t == pl.num_programs(1) - 1)
    def _():
        pex_ref[...] = acc_ref[...] / lencol_ref[...]


def _forward(sent, lengths, params, *, ts=8, bb=64):
    B, T = sent.shape
    V, E = params["emb"].shape
    H = params["wh0"].shape[0]
    Tm1 = T - 1

    Tp = ((Tm1 + ts - 1) // ts) * ts
    Bp = ((B + bb - 1) // bb) * bb
    nb, nt = Bp // bb, Tp // ts

    sent_pad = jnp.pad(sent[:, :Tm1].astype(jnp.int32),
                       ((0, Bp - B), (0, Tp - Tm1)))

    tgt = sent[:, 1:].astype(jnp.int32)
    tgt = jnp.pad(tgt, ((0, Bp - B), (0, Tp - Tm1)))
    tgt = jnp.transpose(
        tgt.reshape(nb, bb, nt, ts), (0, 2, 1, 3)).reshape(nb, nt, bb, ts, 1)

    lenm1 = (lengths - 1).astype(jnp.float32).reshape(B, 1)
    lenm1 = jnp.pad(lenm1, ((0, Bp - B), (0, 0)), constant_values=1.0)
    len3 = lenm1.reshape(nb, bb, 1, 1)

    wl_bf = params["wl"].astype(jnp.bfloat16)

    grid_spec = pltpu.PrefetchScalarGridSpec(
        num_scalar_prefetch=1,
        grid=(nb, nt),
        in_specs=[
            pl.BlockSpec((V, E), lambda b, t, *_: (0, 0)),
            pl.BlockSpec((1, 1, bb, ts, 1),
                         lambda b, t, *_: (b, t, 0, 0, 0)),
            pl.BlockSpec((1, bb, 1, 1),
                         lambda b, t, *_: (b, 0, 0, 0)),
            pl.BlockSpec((bb, 1), lambda b, t, *_: (b, 0)),
            pl.BlockSpec((E, H), lambda b, t, *_: (0, 0)),
            pl.BlockSpec((H, H), lambda b, t, *_: (0, 0)),
            pl.BlockSpec((1, H), lambda b, t, *_: (0, 0)),
            pl.BlockSpec((H, H), lambda b, t, *_: (0, 0)),
            pl.BlockSpec((H, H), lambda b, t, *_: (0, 0)),
            pl.BlockSpec((1, H), lambda b, t, *_: (0, 0)),
            pl.BlockSpec((H, V), lambda b, t, *_: (0, 0)),
            pl.BlockSpec((1, V), lambda b, t, *_: (0, 0)),
        ],
        out_specs=[
            pl.BlockSpec((bb, ts, V), lambda b, t, *_: (b, t, 0)),
            pl.BlockSpec((bb, 1), lambda b, t, *_: (b, 0)),
        ],
        scratch_shapes=[
            pltpu.VMEM((bb, H), jnp.float32),
            pltpu.VMEM((bb, H), jnp.float32),
            pltpu.VMEM((bb, 1), jnp.float32),
            pltpu.VMEM((ts * bb, E), jnp.float32),
        ],
    )

    logits_bm, per_example = pl.pallas_call(
        functools.partial(_rnn_lm_kernel, ts=ts, bb=bb),
        grid_spec=grid_spec,
        out_shape=(
            jax.ShapeDtypeStruct((Bp, Tp, V), jnp.float32),
            jax.ShapeDtypeStruct((Bp, 1), jnp.float32),
        ),
        compiler_params=pltpu.CompilerParams(
            dimension_semantics=("parallel", "arbitrary")),
    )(sent_pad,
      params["emb"], tgt, len3, lenm1,
      params["wx0"], params["wh0"], params["b0"],
      params["wx1"], params["wh1"], params["b1"], wl_bf, params["bl"])

    logits = logits_bm[:B, :Tm1]
    loss = -jnp.mean(per_example[:B, 0])
    return loss, logits


def kernel(sent, lengths, emb, wx0, wh0, b0, wx1, wh1, b1, wl, bl):
    params = {
        "emb": emb,
        "wx0": wx0, "wh0": wh0, "b0": b0,
        "wx1": wx1, "wh1": wh1, "b1": b1,
        "wl": wl, "bl": bl,
    }
    return _forward(sent, lengths, params)


# speedup vs baseline: 1.0356x; 1.0356x over previous
import functools

import jax
import jax.numpy as jnp
from jax import lax
from jax.experimental import pallas as pl
from jax.experimental.pallas import tpu as pltpu


def _rnn_lm_kernel(sent_ref,
                   emb_ref, tgt_ref, len3_ref, lencol_ref,
                   wx0_ref, wh0_ref, b0_ref, wx1_ref, wh1_ref, b1_ref,
                   wl_ref, bl_ref,
                   logits_ref, pex_ref,
                   h0_ref, h1_ref, acc_ref, xall_ref, *, ts, bb):
    b = pl.program_id(0)
    t = pl.program_id(1)
    E = emb_ref.shape[-1]
    H = wh0_ref.shape[0]
    V = wl_ref.shape[1]

    @pl.when(t == 0)
    def _():
        h0_ref[...] = jnp.zeros_like(h0_ref)
        h1_ref[...] = jnp.zeros_like(h1_ref)
        acc_ref[...] = jnp.zeros_like(acc_ref)

    for k in range(ts):
        for i in range(bb):
            tok = sent_ref[b * bb + i, t * ts + k]
            xall_ref[k * bb + i, :] = emb_ref[tok, :]

    xp0 = (jnp.dot(xall_ref[...], wx0_ref[...],
                   preferred_element_type=jnp.float32)
           + b0_ref[...]).reshape(ts, bb, H)

    h0 = h0_ref[...]
    h1 = h1_ref[...]
    hs = []
    for k in range(ts):
        h0 = jnp.tanh(
            xp0[k] + jnp.dot(h0, wh0_ref[...],
                             preferred_element_type=jnp.float32))
        h1 = jnp.tanh(
            jnp.dot(h0, wx1_ref[...], preferred_element_type=jnp.float32)
            + jnp.dot(h1, wh1_ref[...], preferred_element_type=jnp.float32)
            + b1_ref[...])
        hs.append(h1)
    h0_ref[...] = h0
    h1_ref[...] = h1

    h_flat = jnp.swapaxes(jnp.stack(hs, axis=0), 0, 1).reshape(bb * ts, H)

    logits_ref[...] = (jnp.dot(h_flat.astype(jnp.bfloat16), wl_ref[...],
                               preferred_element_type=jnp.float32)
                       + bl_ref[...]).reshape(bb, ts, V)
    logits3 = logits_ref[...]

    tgt = tgt_ref[0, 0]
    vs = min(256, V)
    s = jnp.zeros((bb, ts, 1), jnp.float32)
    g = jnp.zeros((bb, ts, 1), jnp.float32)
    for j in range(V // vs):
        blk = logits3[:, :, j * vs:(j + 1) * vs]
        s = s + jnp.sum(jnp.exp(blk), axis=-1, keepdims=True)
        col = lax.broadcasted_iota(jnp.int32, (bb, ts, vs), 2) + j * vs
        g = g + jnp.sum(jnp.where(col == tgt, blk, 0.0),
                        axis=-1, keepdims=True)
    lse = jnp.log(s)

    k_ids = lax.broadcasted_iota(jnp.int32, (bb, ts, 1), 1)
    step_f = (k_ids + t * ts).astype(jnp.float32)
    masked = jnp.where(step_f < len3_ref[0], g - lse, 0.0)
    acc_ref[...] = acc_ref[...] + jnp.sum(masked, axis=1)

    @pl.when(t == pl.num_programs(1) - 1)
    def _():
        pex_ref[...] = acc_ref[...] / lencol_ref[...]


def _forward(sent, lengths, params, *, ts=8, bb=64):
    B, T = sent.shape
    V, E = params["emb"].shape
    H = params["wh0"].shape[0]
    Tm1 = T - 1

    Tp = ((Tm1 + ts - 1) // ts) * ts
    Bp = ((B + bb - 1) // bb) * bb
    nb, nt = Bp // bb, Tp // ts

    sent_pad = jnp.pad(sent[:, :Tm1].astype(jnp.int32),
                       ((0, Bp - B), (0, Tp - Tm1)))

    tgt = sent[:, 1:].astype(jnp.int32)
    tgt = jnp.pad(tgt, ((0, Bp - B), (0, Tp - Tm1)))
    tgt = jnp.transpose(
        tgt.reshape(nb, bb, nt, ts), (0, 2, 1, 3)).reshape(nb, nt, bb, ts, 1)

    lenm1 = (lengths - 1).astype(jnp.float32).reshape(B, 1)
    lenm1 = jnp.pad(lenm1, ((0, Bp - B), (0, 0)), constant_values=1.0)
    len3 = lenm1.reshape(nb, bb, 1, 1)

    wl_bf = params["wl"].astype(jnp.bfloat16)

    grid_spec = pltpu.PrefetchScalarGridSpec(
        num_scalar_prefetch=1,
        grid=(nb, nt),
        in_specs=[
            pl.BlockSpec((V, E), lambda b, t, *_: (0, 0)),
            pl.BlockSpec((1, 1, bb, ts, 1),
                         lambda b, t, *_: (b, t, 0, 0, 0)),
            pl.BlockSpec((1, bb, 1, 1),
                         lambda b, t, *_: (b, 0, 0, 0)),
            pl.BlockSpec((bb, 1), lambda b, t, *_: (b, 0)),
            pl.BlockSpec((E, H), lambda b, t, *_: (0, 0)),
            pl.BlockSpec((H, H), lambda b, t, *_: (0, 0)),
            pl.BlockSpec((1, H), lambda b, t, *_: (0, 0)),
            pl.BlockSpec((H, H), lambda b, t, *_: (0, 0)),
            pl.BlockSpec((H, H), lambda b, t, *_: (0, 0)),
            pl.BlockSpec((1, H), lambda b, t, *_: (0, 0)),
            pl.BlockSpec((H, V), lambda b, t, *_: (0, 0)),
            pl.BlockSpec((1, V), lambda b, t, *_: (0, 0)),
        ],
        out_specs=[
            pl.BlockSpec((bb, ts, V), lambda b, t, *_: (b, t, 0)),
            pl.BlockSpec((bb, 1), lambda b, t, *_: (b, 0)),
        ],
        scratch_shapes=[
            pltpu.VMEM((bb, H), jnp.float32),
            pltpu.VMEM((bb, H), jnp.float32),
            pltpu.VMEM((bb, 1), jnp.float32),
            pltpu.VMEM((ts * bb, E), jnp.float32),
        ],
    )

    logits_bm, per_example = pl.pallas_call(
        functools.partial(_rnn_lm_kernel, ts=ts, bb=bb),
        grid_spec=grid_spec,
        out_shape=(
            jax.ShapeDtypeStruct((Bp, Tp, V), jnp.float32),
            jax.ShapeDtypeStruct((Bp, 1), jnp.float32),
        ),
        compiler_params=pltpu.CompilerParams(
            dimension_semantics=("parallel", "arbitrary")),
    )(sent_pad,
      params["emb"], tgt, len3, lenm1,
      params["wx0"], params["wh0"], params["b0"],
      params["wx1"], params["wh1"], params["b1"], wl_bf, params["bl"])

    logits = logits_bm[:B, :Tm1]
    loss = -jnp.mean(per_example[:B, 0])
    return loss, logits


def kernel(sent, lengths, emb, wx0, wh0, b0, wx1, wh1, b1, wl, bl):
    params = {
        "emb": emb,
        "wx0": wx0, "wh0": wh0, "b0": b0,
        "wx1": wx1, "wh1": wh1, "b1": b1,
        "wl": wl, "bl": bl,
    }
    return _forward(sent, lengths, params)
